# baseline (speedup 1.0000x reference)
import functools

import jax
import jax.numpy as jnp
from jax import lax
from jax.experimental import pallas as pl
from jax.experimental.pallas import tpu as pltpu
from jax.experimental.pallas import tpu_sc as plsc

N_ROWS = 1000
B_TOTAL = 1024 * 20
NW = 32
B_PER_W = B_TOTAL // NW
CHUNK = 64
N_CHUNKS = B_PER_W // CHUNK
L = 16


def _recip_body(x_ref, o_ref):
    x = x_ref[...]
    d = jnp.where(x == 0.0, jnp.float32(9999999.99), x)
    o_ref[...] = 1.0 / d


_recip_call = pl.pallas_call(
    _recip_body,
    out_shape=jax.ShapeDtypeStruct((N_ROWS, N_ROWS), jnp.float32),
)


_sc_mesh = plsc.VectorSubcoreMesh(core_axis_name="c", subcore_axis_name="s")


@functools.partial(
    pl.kernel,
    mesh=_sc_mesh,
    out_type=jax.ShapeDtypeStruct((B_TOTAL, N_ROWS), jnp.float32),
    compiler_params=pltpu.CompilerParams(use_tc_tiling_on_sc=False),
    scratch_types=[
        pltpu.VMEM((B_PER_W,), jnp.int32),
        pltpu.VMEM((N_CHUNKS, CHUNK), jnp.int32),
        pltpu.VMEM((2, CHUNK, N_ROWS), jnp.float32),
        pltpu.SemaphoreType.DMA,
        pltpu.SemaphoreType.DMA,
        pltpu.SemaphoreType.DMA,
        pltpu.SemaphoreType.DMA,
    ],
)
def _sc_gather(venue_hbm, poi_hbm, table_hbm, out_hbm,
               poi_v, idx_v, rows_v, sem_g0, sem_g1, sem_o0, sem_o1):
    wid = lax.axis_index("s") * 2 + lax.axis_index("c")
    base_w = wid * B_PER_W
    sem_g = (sem_g0, sem_g1)
    sem_o = (sem_o0, sem_o1)

    pltpu.sync_copy(poi_hbm.at[pl.ds(base_w, B_PER_W)], poi_v)
    idx_copies = [
        pltpu.async_copy(venue_hbm.at[poi_v.at[pl.ds(j * CHUNK, CHUNK)]],
                         idx_v.at[j], sem_g0)
        for j in range(N_CHUNKS)
    ]
    for c in idx_copies:
        c.wait()

    gathers = [None] * N_CHUNKS
    outs = [None] * N_CHUNKS
    for j in range(N_CHUNKS):
        p = j % 2
        if j >= 2:
            outs[j - 2].wait()
        gathers[j] = pltpu.async_copy(table_hbm.at[idx_v.at[j]],
                                      rows_v.at[p], sem_g[p])
        if j >= 1:
            gathers[j - 1].wait()
            outs[j - 1] = pltpu.async_copy(
                rows_v.at[(j - 1) % 2],
                out_hbm.at[pl.ds(base_w + (j - 1) * CHUNK, CHUNK)],
                sem_o[(j - 1) % 2])
    gathers[N_CHUNKS - 1].wait()
    outs[N_CHUNKS - 1] = pltpu.async_copy(
        rows_v.at[(N_CHUNKS - 1) % 2],
        out_hbm.at[pl.ds(base_w + (N_CHUNKS - 1) * CHUNK, CHUNK)],
        sem_o[(N_CHUNKS - 1) % 2])
    outs[N_CHUNKS - 2].wait()
    outs[N_CHUNKS - 1].wait()


def kernel(venueid2coor, inputs_poi, poi_distance_matrix):
    recip = _recip_call(poi_distance_matrix)
    poi_flat = inputs_poi.reshape(-1)
    out = _sc_gather(venueid2coor, poi_flat, recip)
    return out.reshape(inputs_poi.shape[0], inputs_poi.shape[1], N_ROWS)
